# baseline (device time: 30314 ns/iter reference)
import jax
import jax.numpy as jnp
from jax import lax
from jax.experimental import pallas as pl
from jax.experimental.pallas import tpu as pltpu

N_DEV = 4
N_LAYERS = 3
H = 512
QH = 128
C = 2
R = 128


def kernel(x, Win0, Wout0, Win1, Wout1, Win2, Wout2):
    b, d_loc = x.shape

    def body(x_ref, win0_ref, wout0_ref, win1_ref, wout1_ref,
             win2_ref, wout2_ref, out_ref,
             win_bf, wout_bf, pq_buf, rs_recv, hfull,
             rs_send_sems, rs_recv_sems, ag_send_sems, ag_recv_sems):
        my = lax.axis_index("i")
        wins = [win0_ref, win1_ref, win2_ref]
        wouts = [wout0_ref, wout1_ref, wout2_ref]

        barrier_sem = pltpu.get_barrier_semaphore()
        for k in range(N_DEV - 1):
            pl.semaphore_signal(
                barrier_sem, inc=1,
                device_id=(lax.rem(my + 1 + k, N_DEV),),
                device_id_type=pl.DeviceIdType.MESH,
            )
        pl.semaphore_wait(barrier_sem, N_DEV - 1)

        for l in range(N_LAYERS):
            win_bf[l] = wins[l][...].astype(jnp.bfloat16)
            wout_bf[l] = wouts[l][...].astype(jnp.bfloat16)

        def start_rs(c, l, x_cur):
            partial = jnp.dot(
                x_cur.astype(jnp.bfloat16), win_bf[l],
                preferred_element_type=jnp.float32,
            )
            pbf = partial.astype(jnp.bfloat16)
            for q in range(N_DEV):
                pq_buf[c, q] = pbf[:, q * QH:(q + 1) * QH]
            rdmas = []
            for k in range(N_DEV - 1):
                peer = lax.rem(my + 1 + k, N_DEV)
                rdma = pltpu.make_async_remote_copy(
                    src_ref=pq_buf.at[c, peer],
                    dst_ref=rs_recv.at[c, N_DEV - 2 - k],
                    send_sem=rs_send_sems.at[c, k],
                    recv_sem=rs_recv_sems.at[c, N_DEV - 2 - k],
                    device_id=(peer,),
                    device_id_type=pl.DeviceIdType.MESH,
                )
                rdma.start()
                rdmas.append(rdma)
            own = pq_buf[c, my].astype(jnp.float32)
            return rdmas, own

        def reduce_and_start_ag(c, l, rs_rdmas, own):
            for rdma in rs_rdmas:
                rdma.wait()
            hq = jnp.maximum(
                own
                + rs_recv[c, 0].astype(jnp.float32)
                + rs_recv[c, 1].astype(jnp.float32)
                + rs_recv[c, 2].astype(jnp.float32),
                0.0,
            )
            hfull[c, my] = hq.astype(jnp.bfloat16)
            ag_rdmas = []
            for k in range(N_DEV - 1):
                peer = lax.rem(my + 1 + k, N_DEV)
                rdma = pltpu.make_async_remote_copy(
                    src_ref=hfull.at[c, my],
                    dst_ref=hfull.at[c, my],
                    send_sem=ag_send_sems.at[c, k],
                    recv_sem=ag_recv_sems.at[c, N_DEV - 2 - k],
                    device_id=(peer,),
                    device_id_type=pl.DeviceIdType.MESH,
                )
                rdma.start()
                ag_rdmas.append(rdma)
            return ag_rdmas

        def finish_ag(c, l, ag_rdmas):
            for rdma in ag_rdmas:
                rdma.wait()
            hcat = jnp.concatenate(
                [hfull[c, q] for q in range(N_DEV)], axis=1
            )
            return jnp.dot(
                hcat, wout_bf[l], preferred_element_type=jnp.float32
            )

        xs = [x_ref[c * R:(c + 1) * R, :] for c in range(C)]
        st = [start_rs(c, 0, xs[c]) for c in range(C)]
        for l in range(N_LAYERS):
            mids = [reduce_and_start_ag(c, l, *st[c]) for c in range(C)]
            for c in range(C):
                xc = finish_ag(c, l, mids[c])
                if l < N_LAYERS - 1:
                    st[c] = start_rs(c, l + 1, xc)
                else:
                    out_ref[c * R:(c + 1) * R, :] = xc

    return pl.pallas_call(
        body,
        out_shape=jax.ShapeDtypeStruct((b, d_loc), jnp.float32),
        in_specs=[pl.BlockSpec(memory_space=pltpu.VMEM)] * 7,
        out_specs=pl.BlockSpec(memory_space=pltpu.VMEM),
        scratch_shapes=[
            pltpu.VMEM((N_LAYERS, d_loc, H), jnp.bfloat16),
            pltpu.VMEM((N_LAYERS, H, d_loc), jnp.bfloat16),
            pltpu.VMEM((C, N_DEV, R, QH), jnp.bfloat16),
            pltpu.VMEM((C, N_DEV - 1, R, QH), jnp.bfloat16),
            pltpu.VMEM((C, N_DEV, R, QH), jnp.bfloat16),
            pltpu.SemaphoreType.DMA((C, N_DEV - 1)),
            pltpu.SemaphoreType.DMA((C, N_DEV - 1)),
            pltpu.SemaphoreType.DMA((C, N_DEV - 1)),
            pltpu.SemaphoreType.DMA((C, N_DEV - 1)),
        ],
        compiler_params=pltpu.CompilerParams(collective_id=0),
    )(x, Win0, Wout0, Win1, Wout1, Win2, Wout2)


# device time: 29862 ns/iter; 1.0151x vs baseline; 1.0151x over previous
import jax
import jax.numpy as jnp
from jax import lax
from jax.experimental import pallas as pl
from jax.experimental.pallas import tpu as pltpu

N_DEV = 4
N_LAYERS = 3
H = 512
QH = 128
C = 8
R = 32


def kernel(x, Win0, Wout0, Win1, Wout1, Win2, Wout2):
    b, d_loc = x.shape

    def body(x_ref, win0_ref, wout0_ref, win1_ref, wout1_ref,
             win2_ref, wout2_ref, out_ref,
             win_bf, wout_bf, pq_buf, rs_recv, hfull,
             rs_send_sems, rs_recv_sems, ag_send_sems, ag_recv_sems):
        my = lax.axis_index("i")
        wins = [win0_ref, win1_ref, win2_ref]
        wouts = [wout0_ref, wout1_ref, wout2_ref]

        barrier_sem = pltpu.get_barrier_semaphore()
        for k in range(N_DEV - 1):
            pl.semaphore_signal(
                barrier_sem, inc=1,
                device_id=(lax.rem(my + 1 + k, N_DEV),),
                device_id_type=pl.DeviceIdType.MESH,
            )
        pl.semaphore_wait(barrier_sem, N_DEV - 1)

        for l in range(N_LAYERS):
            win_bf[l] = wins[l][...].astype(jnp.bfloat16)
            wout_bf[l] = wouts[l][...].astype(jnp.bfloat16)

        def start_rs(c, l, x_cur):
            partial = jnp.dot(
                x_cur.astype(jnp.bfloat16), win_bf[l],
                preferred_element_type=jnp.float32,
            )
            pbf = partial.astype(jnp.bfloat16)
            for q in range(N_DEV):
                pq_buf[c, q] = pbf[:, q * QH:(q + 1) * QH]
            rdmas = []
            for k in range(N_DEV - 1):
                peer = lax.rem(my + 1 + k, N_DEV)
                rdma = pltpu.make_async_remote_copy(
                    src_ref=pq_buf.at[c, peer],
                    dst_ref=rs_recv.at[c, N_DEV - 2 - k],
                    send_sem=rs_send_sems.at[c, k],
                    recv_sem=rs_recv_sems.at[c, N_DEV - 2 - k],
                    device_id=(peer,),
                    device_id_type=pl.DeviceIdType.MESH,
                )
                rdma.start()
                rdmas.append(rdma)
            own = pq_buf[c, my].astype(jnp.float32)
            return rdmas, own

        def reduce_and_start_ag(c, l, rs_rdmas, own):
            for rdma in rs_rdmas:
                rdma.wait()
            hq = jnp.maximum(
                own
                + rs_recv[c, 0].astype(jnp.float32)
                + rs_recv[c, 1].astype(jnp.float32)
                + rs_recv[c, 2].astype(jnp.float32),
                0.0,
            )
            hfull[c, my] = hq.astype(jnp.bfloat16)
            ag_rdmas = []
            for k in range(N_DEV - 1):
                peer = lax.rem(my + 1 + k, N_DEV)
                rdma = pltpu.make_async_remote_copy(
                    src_ref=hfull.at[c, my],
                    dst_ref=hfull.at[c, my],
                    send_sem=ag_send_sems.at[c, k],
                    recv_sem=ag_recv_sems.at[c, N_DEV - 2 - k],
                    device_id=(peer,),
                    device_id_type=pl.DeviceIdType.MESH,
                )
                rdma.start()
                ag_rdmas.append(rdma)
            return ag_rdmas

        def finish_ag(c, l, ag_rdmas):
            for rdma in ag_rdmas:
                rdma.wait()
            hcat = jnp.concatenate(
                [hfull[c, q] for q in range(N_DEV)], axis=1
            )
            return jnp.dot(
                hcat, wout_bf[l], preferred_element_type=jnp.float32
            )

        xs = [x_ref[c * R:(c + 1) * R, :] for c in range(C)]
        st = [start_rs(c, 0, xs[c]) for c in range(C)]
        for l in range(N_LAYERS):
            mids = [reduce_and_start_ag(c, l, *st[c]) for c in range(C)]
            for c in range(C):
                xc = finish_ag(c, l, mids[c])
                if l < N_LAYERS - 1:
                    st[c] = start_rs(c, l + 1, xc)
                else:
                    out_ref[c * R:(c + 1) * R, :] = xc

    return pl.pallas_call(
        body,
        out_shape=jax.ShapeDtypeStruct((b, d_loc), jnp.float32),
        in_specs=[pl.BlockSpec(memory_space=pltpu.VMEM)] * 7,
        out_specs=pl.BlockSpec(memory_space=pltpu.VMEM),
        scratch_shapes=[
            pltpu.VMEM((N_LAYERS, d_loc, H), jnp.bfloat16),
            pltpu.VMEM((N_LAYERS, H, d_loc), jnp.bfloat16),
            pltpu.VMEM((C, N_DEV, R, QH), jnp.bfloat16),
            pltpu.VMEM((C, N_DEV - 1, R, QH), jnp.bfloat16),
            pltpu.VMEM((C, N_DEV, R, QH), jnp.bfloat16),
            pltpu.SemaphoreType.DMA((C, N_DEV - 1)),
            pltpu.SemaphoreType.DMA((C, N_DEV - 1)),
            pltpu.SemaphoreType.DMA((C, N_DEV - 1)),
            pltpu.SemaphoreType.DMA((C, N_DEV - 1)),
        ],
        compiler_params=pltpu.CompilerParams(collective_id=0),
    )(x, Win0, Wout0, Win1, Wout1, Win2, Wout2)


# device time: 29006 ns/iter; 1.0451x vs baseline; 1.0295x over previous
import jax
import jax.numpy as jnp
from jax import lax
from jax.experimental import pallas as pl
from jax.experimental.pallas import tpu as pltpu

N_DEV = 4
N_LAYERS = 3
H = 512
QH = 128
C = 4
R = 64


def kernel(x, Win0, Wout0, Win1, Wout1, Win2, Wout2):
    b, d_loc = x.shape

    def body(x_ref, win0_ref, wout0_ref, win1_ref, wout1_ref,
             win2_ref, wout2_ref, out_ref,
             win_bf, wout_bf, winstage, woutstage,
             win_copy_sems, wout_copy_sems,
             pq_buf, rs_recv, hfull,
             rs_send_sems, rs_recv_sems, ag_send_sems, ag_recv_sems):
        my = lax.axis_index("i")

        wout_copies = []
        for i, ref in enumerate([wout0_ref, wout1_ref, wout2_ref]):
            cp = pltpu.make_async_copy(
                ref, woutstage.at[i], wout_copy_sems.at[i]
            )
            cp.start()
            wout_copies.append(cp)
        win_copies = []
        for i, ref in enumerate([win1_ref, win2_ref]):
            cp = pltpu.make_async_copy(
                ref, winstage.at[i], win_copy_sems.at[i]
            )
            cp.start()
            win_copies.append(cp)

        barrier_sem = pltpu.get_barrier_semaphore()
        for k in range(N_DEV - 1):
            pl.semaphore_signal(
                barrier_sem, inc=1,
                device_id=(lax.rem(my + 1 + k, N_DEV),),
                device_id_type=pl.DeviceIdType.MESH,
            )
        pl.semaphore_wait(barrier_sem, N_DEV - 1)

        win_bf[0] = win0_ref[...].astype(jnp.bfloat16)

        def start_rs(c, l, x_cur):
            partial = jnp.dot(
                x_cur.astype(jnp.bfloat16), win_bf[l],
                preferred_element_type=jnp.float32,
            )
            pbf = partial.astype(jnp.bfloat16)
            for q in range(N_DEV):
                pq_buf[c, q] = pbf[:, q * QH:(q + 1) * QH]
            rdmas = []
            for k in range(N_DEV - 1):
                peer = lax.rem(my + 1 + k, N_DEV)
                rdma = pltpu.make_async_remote_copy(
                    src_ref=pq_buf.at[c, peer],
                    dst_ref=rs_recv.at[c, N_DEV - 2 - k],
                    send_sem=rs_send_sems.at[c, k],
                    recv_sem=rs_recv_sems.at[c, N_DEV - 2 - k],
                    device_id=(peer,),
                    device_id_type=pl.DeviceIdType.MESH,
                )
                rdma.start()
                rdmas.append(rdma)
            own = pq_buf[c, my].astype(jnp.float32)
            return rdmas, own

        def reduce_and_start_ag(c, l, rs_rdmas, own):
            for rdma in rs_rdmas:
                rdma.wait()
            hq = jnp.maximum(
                own
                + rs_recv[c, 0].astype(jnp.float32)
                + rs_recv[c, 1].astype(jnp.float32)
                + rs_recv[c, 2].astype(jnp.float32),
                0.0,
            )
            hfull[c, my] = hq.astype(jnp.bfloat16)
            ag_rdmas = []
            for k in range(N_DEV - 1):
                peer = lax.rem(my + 1 + k, N_DEV)
                rdma = pltpu.make_async_remote_copy(
                    src_ref=hfull.at[c, my],
                    dst_ref=hfull.at[c, my],
                    send_sem=ag_send_sems.at[c, k],
                    recv_sem=ag_recv_sems.at[c, N_DEV - 2 - k],
                    device_id=(peer,),
                    device_id_type=pl.DeviceIdType.MESH,
                )
                rdma.start()
                ag_rdmas.append(rdma)
            return ag_rdmas

        def finish_ag(c, l, ag_rdmas):
            for rdma in ag_rdmas:
                rdma.wait()
            hcat = jnp.concatenate(
                [hfull[c, q] for q in range(N_DEV)], axis=1
            )
            return jnp.dot(
                hcat, wout_bf[l], preferred_element_type=jnp.float32
            )

        xs = [x_ref[c * R:(c + 1) * R, :] for c in range(C)]
        st = [start_rs(c, 0, xs[c]) for c in range(C)]
        for l in range(N_LAYERS):
            mids = [reduce_and_start_ag(c, l, *st[c]) for c in range(C)]
            wout_copies[l].wait()
            wout_bf[l] = woutstage[l].astype(jnp.bfloat16)
            if l < N_LAYERS - 1:
                win_copies[l].wait()
                win_bf[l + 1] = winstage[l].astype(jnp.bfloat16)
            for c in range(C):
                xc = finish_ag(c, l, mids[c])
                if l < N_LAYERS - 1:
                    st[c] = start_rs(c, l + 1, xc)
                else:
                    out_ref[c * R:(c + 1) * R, :] = xc

    return pl.pallas_call(
        body,
        out_shape=jax.ShapeDtypeStruct((b, d_loc), jnp.float32),
        in_specs=(
            [pl.BlockSpec(memory_space=pltpu.VMEM)] * 2
            + [pl.BlockSpec(memory_space=pl.ANY)] * 5
        ),
        out_specs=pl.BlockSpec(memory_space=pltpu.VMEM),
        scratch_shapes=[
            pltpu.VMEM((N_LAYERS, d_loc, H), jnp.bfloat16),
            pltpu.VMEM((N_LAYERS, H, d_loc), jnp.bfloat16),
            pltpu.VMEM((2, d_loc, H), jnp.float32),
            pltpu.VMEM((N_LAYERS, H, d_loc), jnp.float32),
            pltpu.SemaphoreType.DMA((2,)),
            pltpu.SemaphoreType.DMA((N_LAYERS,)),
            pltpu.VMEM((C, N_DEV, R, QH), jnp.bfloat16),
            pltpu.VMEM((C, N_DEV - 1, R, QH), jnp.bfloat16),
            pltpu.VMEM((C, N_DEV, R, QH), jnp.bfloat16),
            pltpu.SemaphoreType.DMA((C, N_DEV - 1)),
            pltpu.SemaphoreType.DMA((C, N_DEV - 1)),
            pltpu.SemaphoreType.DMA((C, N_DEV - 1)),
            pltpu.SemaphoreType.DMA((C, N_DEV - 1)),
        ],
        compiler_params=pltpu.CompilerParams(collective_id=0),
    )(x, Win0, Wout0, Win1, Wout1, Win2, Wout2)
